# Initial kernel scaffold; baseline (speedup 1.0000x reference)
import jax
import jax.numpy as jnp
from jax.experimental import pallas as pl


def kernel(pre_x, x, edge_index, edge_type, num_prop, num_category, des_tensor, tweet_tensor, params):
    raise NotImplementedError("write your pallas kernel here")


# speedup vs baseline: 12.2936x; 12.2936x over previous
import functools

import numpy as np
import jax
import jax.numpy as jnp
from jax import lax
from jax.experimental import pallas as pl
from jax.experimental.pallas import tpu as pltpu
from jax.experimental.pallas import tpu_sc as plsc

N = 10000
E = 320000
HID = 160
LM = 768

NC = 2
NS = 16
NW = NC * NS
EW = E // NW
CH = 80
NCHUNK = EW // CH
NP = N
TPT = 640
TAIL = N - (NS - 1) * TPT
NEG = np.float32(-3.0e38)

_F32 = jnp.float32


def _lrelu(v):
    return jnp.where(v >= 0, v, 0.01 * v)


def _dotT(a, b):
    return lax.dot_general(a, b, (((1,), (1,)), ((), ())),
                           preferred_element_type=_F32)


def _front_body(nc_ref, des_ref, tw_ref, px_ref, Wnc, bnc, Wd, bd, Wt, bt,
                Wp, bp, Win, bi, h_ref):
    a = _lrelu(_dotT(nc_ref[...], Wnc[...]) + bnc[...])
    d = _lrelu(_dotT(des_ref[...], Wd[...]) + bd[...])
    t = _lrelu(_dotT(tw_ref[...], Wt[...]) + bt[...])
    p = _lrelu(_dotT(px_ref[...], Wp[...]) + bp[...])
    hcat = jnp.concatenate([a, d, t, p], axis=1)
    h_ref[...] = _lrelu(_dotT(hcat, Win[...]) + bi[...])


def _proj_body(h_ref, Wkqv, bkqv, Wkrel, Wvrel, q_ref, kc_ref, vc_ref):
    h = h_ref[...]
    kqv = _dotT(h, Wkqv[...]) + bkqv[...]
    k = kqv[:, 0:HID]
    q = kqv[:, HID:2 * HID]
    v = kqv[:, 2 * HID:3 * HID]
    q_ref[...] = q
    kc_ref[0] = _dotT(k, Wkrel[0])
    kc_ref[1] = _dotT(k, Wkrel[1])
    vc_ref[0] = _dotT(v, Wvrel[0])
    vc_ref[1] = _dotT(v, Wvrel[1])


_INV_SQRT2 = float(1.0 / np.sqrt(2.0))


def _gelu(x):
    return 0.5 * x * (1.0 + lax.erf(x * _INV_SQRT2))


def _comb1_body(h_ref, agg_ref, s_ref, Wo, bo, gv, cv, h1_ref):
    r = 1.0 / (s_ref[...] + 1e-16)
    agg = (agg_ref[0] + agg_ref[1]) * r
    gl = _gelu(agg)
    out = _dotT(gl, Wo[...]) + bo[...]
    h1_ref[...] = out * gv[...] + h_ref[...] * cv[...]


def _comb2_body(h_ref, agg_ref, s_ref, Wo, bo, gv, cv, W1, b1, W2, b2,
                out_ref, em_ref):
    r = 1.0 / (s_ref[...] + 1e-16)
    agg = (agg_ref[0] + agg_ref[1]) * r
    gl = _gelu(agg)
    out = _dotT(gl, Wo[...]) + bo[...]
    h2 = out * gv[...] + h_ref[...] * cv[...]
    em = _lrelu(_dotT(h2, W1[...]) + b1[...])
    em_ref[...] = em
    out_ref[...] = _dotT(em, W2[...]) + b2[...]


def _full(shape):
    nd = len(shape)
    return pl.BlockSpec(shape, lambda i, _n=nd: (0,) * _n)


def _rows(bshape):
    nd = len(bshape)
    if nd == 2:
        return pl.BlockSpec(bshape, lambda i: (i, 0))
    return pl.BlockSpec(bshape, lambda i: (0, i, 0))


def _tc_front(ncat, des, tw, px, Wnc, bnc, Wd, bd, Wt, bt, Wp, bp, Win, bi):
    R = 400
    grid = (N // R,)
    ins = [
        _rows((R, 17)), _rows((R, LM)), _rows((R, LM)), _rows((R, LM)),
        _full(Wnc.shape), _full(bnc.shape), _full(Wd.shape), _full(bd.shape),
        _full(Wt.shape), _full(bt.shape), _full(Wp.shape), _full(bp.shape),
        _full(Win.shape), _full(bi.shape),
    ]
    return pl.pallas_call(
        _front_body, grid=grid, in_specs=ins,
        out_specs=_rows((R, HID)),
        out_shape=jax.ShapeDtypeStruct((N, HID), _F32),
    )(ncat, des, tw, px, Wnc, bnc, Wd, bd, Wt, bt, Wp, bp, Win, bi)


def _tc_proj(h, Wkqv, bkqv, Wkrel, Wvrel):
    R = 1000
    grid = (N // R,)
    ins = [_rows((R, HID))] + [_full(w.shape) for w in
                               (Wkqv, bkqv, Wkrel, Wvrel)]
    outs = (
        pl.BlockSpec((R, HID), lambda i: (i, 0)),
        pl.BlockSpec((2, R, HID), lambda i: (0, i, 0)),
        pl.BlockSpec((2, R, HID), lambda i: (0, i, 0)),
    )
    oshape = (
        jax.ShapeDtypeStruct((N, HID), _F32),
        jax.ShapeDtypeStruct((2, N, HID), _F32),
        jax.ShapeDtypeStruct((2, N, HID), _F32),
    )
    return pl.pallas_call(
        _proj_body, grid=grid, in_specs=ins, out_specs=outs, out_shape=oshape,
    )(h, Wkqv, bkqv, Wkrel, Wvrel)


def _tc_comb1(h, aggP, s01, Wo, bo, gv, cv):
    R = 1000
    grid = (N // R,)
    ins = [
        _rows((R, HID)), _rows((2, R, HID)),
        pl.BlockSpec((R, 1), lambda i: (i, 0)),
        _full(Wo.shape), _full(bo.shape), _full(gv.shape), _full(cv.shape),
    ]
    return pl.pallas_call(
        _comb1_body, grid=grid, in_specs=ins,
        out_specs=_rows((R, HID)),
        out_shape=jax.ShapeDtypeStruct((N, HID), _F32),
    )(h, aggP, s01, Wo, bo, gv, cv)


def _tc_comb2(h, aggP, s01, Wo, bo, gv, cv, W1, b1, W2, b2):
    R = 1000
    grid = (N // R,)
    ins = [
        _rows((R, HID)), _rows((2, R, HID)),
        pl.BlockSpec((R, 1), lambda i: (i, 0)),
        _full(Wo.shape), _full(bo.shape), _full(gv.shape), _full(cv.shape),
        _full(W1.shape), _full(b1.shape), _full(W2.shape), _full(b2.shape),
    ]
    outs = (
        pl.BlockSpec((R, 2), lambda i: (i, 0)),
        pl.BlockSpec((R, 80), lambda i: (i, 0)),
    )
    oshape = (
        jax.ShapeDtypeStruct((N, 2), _F32),
        jax.ShapeDtypeStruct((N, 80), _F32),
    )
    return pl.pallas_call(
        _comb2_body, grid=grid, in_specs=ins, out_specs=outs, out_shape=oshape,
    )(h, aggP, s01, Wo, bo, gv, cv, W1, b1, W2, b2)


_MESH = plsc.VectorSubcoreMesh(core_axis_name="c", subcore_axis_name="s",
                               num_cores=NC, num_subcores=NS)
_SC_PARAMS = pltpu.CompilerParams(needs_layout_passes=False,
                                  use_tc_tiling_on_sc=False)


def _load_idx(src_hbm, dst_hbm, et_hbm, base, dstv, srcv, etv, esrcv):
    pltpu.sync_copy(dst_hbm.at[pl.ds(base, CH)], dstv)
    pltpu.sync_copy(src_hbm.at[pl.ds(base, CH)], srcv)
    pltpu.sync_copy(et_hbm.at[pl.ds(base, CH)], etv)
    for g in range(CH // 16):
        sl = pl.ds(g * 16, 16)
        esrcv[sl] = etv[sl] * N + srcv[sl]


_DSCALE = float(1.0 / np.sqrt(HID))


def _p1_body(q_hbm, k_hbm, src_hbm, dst_hbm, et_hbm, prel_hbm, a_out, m_out,
             dstv, srcv, etv, esrcv, qrows, krows, abuf, matb, mloc, prelv,
             sem1, sem2):
    c = lax.axis_index("c")
    s = lax.axis_index("s")
    wid = c * NS + s

    pltpu.sync_copy(prel_hbm, prelv)
    prl = prelv[pl.ds(0, 16)]
    pr0 = prl[0] * _DSCALE
    pr1 = prl[1] * _DSCALE

    def initm(i, _):
        mloc[pl.ds(i * 16, 16)] = jnp.full((16,), NEG, _F32)
        return _
    lax.fori_loop(0, NP // 16, initm, None)

    def chunk(j, _):
        base = wid * EW + j * CH
        _load_idx(src_hbm, dst_hbm, et_hbm, base, dstv, srcv, etv, esrcv)
        cp1 = pltpu.async_copy(q_hbm.at[dstv], qrows, sem1)
        cp2 = pltpu.async_copy(k_hbm.at[esrcv], krows, sem2)
        cp1.wait()
        cp2.wait()
        iota16 = lax.iota(jnp.int32, 16)
        for g in range(CH // 16):
            for j in range(16):
                i = g * 16 + j
                acc = qrows[i, pl.ds(0, 16)] * krows[i, pl.ds(0, 16)]
                for db in range(1, HID // 16):
                    sl = pl.ds(db * 16, 16)
                    acc = acc + qrows[i, sl] * krows[i, sl]
                matb[pl.ds(j * 16, 16)] = acc
            r = plsc.load_gather(matb, [iota16 * 16])
            for l in range(1, 16):
                r = r + plsc.load_gather(matb, [iota16 * 16 + l])
            etf = etv[pl.ds(g * 16, 16)].astype(_F32)
            abuf[pl.ds(g * 16, 16)] = r * (pr0 + etf * (pr1 - pr0))
        for g in range(CH // 16):
            sl = pl.ds(g * 16, 16)
            dvec = dstv[sl]
            avec = abuf[sl]

            def wcond(done):
                return ~jnp.all(done)

            def wbody(done):
                cur = plsc.load_gather(mloc, [dvec])
                new = jnp.maximum(cur, avec)
                plsc.store_scatter(mloc, [dvec], new, mask=~done)
                re = plsc.load_gather(mloc, [dvec])
                return re >= avec
            lax.while_loop(wcond, wbody, jnp.zeros((16,), jnp.bool_))
        pltpu.sync_copy(abuf, a_out.at[pl.ds(base, CH)])
        return _
    lax.fori_loop(0, NCHUNK, chunk, None)

    pltpu.sync_copy(mloc, m_out.at[pl.ds(wid * NP, NP)])


_MPC = 2000


def _p2_body(a_hbm, m_hbm, v_hbm, src_hbm, dst_hbm, et_hbm,
             agg_out, s_out,
             dstv, esrcv, etmp, arow, vrows, mloc, mtmp, aggsh, ssh, sem1):
    c = lax.axis_index("c")
    s = lax.axis_index("s")
    wid = c * NS + s

    pltpu.sync_copy(m_hbm.at[pl.ds(0, NP)], mloc)

    def mpart(t, _):
        def piece(p, __):
            pltpu.sync_copy(m_hbm.at[pl.ds(t * NP + p * _MPC, _MPC)], mtmp)

            def mmax(i, ___):
                sl = pl.ds(p * _MPC + i * 16, 16)
                mloc[sl] = jnp.maximum(mloc[sl], mtmp[pl.ds(i * 16, 16)])
                return ___
            lax.fori_loop(0, _MPC // 16, mmax, None)
            return __
        lax.fori_loop(0, NP // _MPC, piece, None)
        return _
    lax.fori_loop(1, NW, mpart, None)

    for g in range(CH // 16):
        arow[pl.ds(g * 16, 16)] = jnp.zeros((16,), _F32)

    def zb(i, _):
        for db in range(HID // 16):
            vrows[i, pl.ds(db * 16, 16)] = jnp.zeros((16,), _F32)
        return _
    lax.fori_loop(0, CH, zb, None)
    nr = N // NS
    for b in range(nr // CH):
        pltpu.sync_copy(vrows, aggsh.at[pl.ds(s * nr + b * CH, CH)])
    rem = nr % CH
    pltpu.sync_copy(vrows.at[pl.ds(0, rem)],
                    aggsh.at[pl.ds(s * nr + (nr // CH) * CH, rem)])

    @pl.when(s < NS - 1)
    def _zs_full():
        for b in range(TPT // CH):
            pltpu.sync_copy(arow, ssh.at[pl.ds(s * TPT + b * CH, CH)])

    @pl.when(s == NS - 1)
    def _zs_tail():
        for b in range(TAIL // CH):
            pltpu.sync_copy(arow, ssh.at[pl.ds((NS - 1) * TPT + b * CH, CH)])
    plsc.subcore_barrier()

    def chunk(j, _):
        base = wid * EW + j * CH
        pltpu.sync_copy(dst_hbm.at[pl.ds(base, CH)], dstv)
        pltpu.sync_copy(src_hbm.at[pl.ds(base, CH)], esrcv)
        pltpu.sync_copy(et_hbm.at[pl.ds(base, CH)], etmp)
        pltpu.sync_copy(a_hbm.at[pl.ds(base, CH)], arow)
        for g in range(CH // 16):
            sl = pl.ds(g * 16, 16)
            esrcv[sl] = etmp[sl] * N + esrcv[sl]
        cp = pltpu.async_copy(v_hbm.at[esrcv], vrows, sem1)
        for g in range(CH // 16):
            sl = pl.ds(g * 16, 16)
            mv = plsc.load_gather(mloc, [dstv[sl]])
            arow[sl] = jnp.exp(arow[sl] - mv)
        pltpu.sync_copy(arow, ssh.at[dstv], add=True)
        cp.wait()
        for g in range(CH // 16):
            evg = arow[pl.ds(g * 16, 16)]
            for j in range(16):
                i = g * 16 + j
                e1 = evg[j]
                for db in range(HID // 16):
                    sl2 = pl.ds(db * 16, 16)
                    vrows[i, sl2] = vrows[i, sl2] * e1
        pltpu.sync_copy(vrows, aggsh.at[dstv], add=True)
        return _
    lax.fori_loop(0, NCHUNK, chunk, None)
    plsc.subcore_barrier()

    pltpu.sync_copy(aggsh.at[pl.ds(s * nr, nr)],
                    agg_out.at[pl.ds(c * N + s * nr, nr)])

    @pl.when(s < NS - 1)
    def _s_full():
        pltpu.sync_copy(ssh.at[pl.ds(s * TPT, TPT)],
                        s_out.at[pl.ds(c * N + s * TPT, TPT)])

    @pl.when(s == NS - 1)
    def _s_tail():
        pltpu.sync_copy(ssh.at[pl.ds((NS - 1) * TPT, TAIL)],
                        s_out.at[pl.ds(c * N + (NS - 1) * TPT, TAIL)])


_sc_pass1 = pl.kernel(
    _p1_body,
    out_type=(
        jax.ShapeDtypeStruct((E,), _F32),
        jax.ShapeDtypeStruct((NW * NP,), _F32),
    ),
    mesh=_MESH,
    scratch_types=[
        pltpu.VMEM((CH,), jnp.int32),
        pltpu.VMEM((CH,), jnp.int32),
        pltpu.VMEM((CH,), jnp.int32),
        pltpu.VMEM((CH,), jnp.int32),
        pltpu.VMEM((CH, HID), _F32),
        pltpu.VMEM((CH, HID), _F32),
        pltpu.VMEM((CH,), _F32),
        pltpu.VMEM((256,), _F32),
        pltpu.VMEM((NP,), _F32),
        pltpu.VMEM((16,), _F32),
        pltpu.SemaphoreType.DMA,
        pltpu.SemaphoreType.DMA,
    ],
    compiler_params=_SC_PARAMS,
)

_sc_pass2 = pl.kernel(
    _p2_body,
    out_type=(
        jax.ShapeDtypeStruct((NC * N, HID), _F32),
        jax.ShapeDtypeStruct((NC * N,), _F32),
    ),
    mesh=_MESH,
    scratch_types=[
        pltpu.VMEM((CH,), jnp.int32),
        pltpu.VMEM((CH,), jnp.int32),
        pltpu.VMEM((CH,), jnp.int32),
        pltpu.VMEM((CH,), _F32),
        pltpu.VMEM((CH, HID), _F32),
        pltpu.VMEM((NP,), _F32),
        pltpu.VMEM((_MPC,), _F32),
        pltpu.VMEM_SHARED((N, HID), _F32),
        pltpu.VMEM_SHARED((N,), _F32),
        pltpu.SemaphoreType.DMA,
    ],
    compiler_params=_SC_PARAMS,
)


def kernel(pre_x, x, edge_index, edge_type, num_prop, num_category,
           des_tensor, tweet_tensor, params):
    p = params
    f32 = _F32
    dscale = float(1.0 / np.sqrt(HID))

    Wnc = jnp.concatenate([
        jnp.concatenate([p['Wn'], jnp.zeros((32, 11), f32)], axis=1),
        jnp.concatenate([jnp.zeros((32, 6), f32), p['Wc']], axis=1),
    ], axis=0)
    bnc = jnp.concatenate([p['bn'], p['bc']])[None, :]
    ncat = jnp.concatenate([num_prop, num_category], axis=1)

    h = _tc_front(ncat, des_tensor, tweet_tensor, pre_x,
                  Wnc, bnc, p['Wd'], p['bd'][None, :], p['Wt'],
                  p['bt'][None, :], p['Wp'], p['bp'][None, :],
                  p['Win'], p['bin'][None, :])

    out = em = None
    src = edge_index[0]
    dst = edge_index[1]
    for i in (1, 2):
        Wkqv = p['Wkqv%d' % i]
        bkqv = p['bkqv%d' % i][None, :]
        Wkrel = p['Wkrel%d' % i]
        Wvrel = p['Wvrel%d' % i]
        prel16 = jnp.pad(p['prel%d' % i], (0, 14))
        g = jax.nn.sigmoid(p['skip%d' % i])
        gv = (g * jnp.ones((HID,), f32))[None, :]
        cv = ((1.0 - g) * jnp.ones((HID,), f32))[None, :]

        q, kc, vc = _tc_proj(h, Wkqv, bkqv, Wkrel, Wvrel)
        a, mP = _sc_pass1(q, kc.reshape(2 * N, HID), src, dst, edge_type,
                          prel16)
        aggF, sP = _sc_pass2(a, mP, vc.reshape(2 * N, HID),
                             src, dst, edge_type)
        aggP = aggF.reshape(NC, N, HID)
        s01 = (sP[:N] + sP[N:2 * N])[:, None]
        if i == 1:
            h = _tc_comb1(h, aggP, s01, p['Wout%d' % i],
                          p['bout%d' % i][None, :], gv, cv)
        else:
            out, em = _tc_comb2(h, aggP, s01, p['Wout%d' % i],
                                p['bout%d' % i][None, :], gv, cv,
                                p['W1'], p['b1'][None, :],
                                p['W2'], p['b2'][None, :])
    return out, em


# speedup vs baseline: 17.3326x; 1.4099x over previous
import functools

import numpy as np
import jax
import jax.numpy as jnp
from jax import lax
from jax.experimental import pallas as pl
from jax.experimental.pallas import tpu as pltpu
from jax.experimental.pallas import tpu_sc as plsc

N = 10000
E = 320000
HID = 160
LM = 768

NC = 2
NS = 16
NW = NC * NS
EW = E // NW
CH = 80
NCHUNK = EW // CH
NP = N
TPT = 640
TAIL = N - (NS - 1) * TPT
NEG = np.float32(-3.0e38)

_F32 = jnp.float32


def _lrelu(v):
    return jnp.where(v >= 0, v, 0.01 * v)


def _dotT(a, b):
    return lax.dot_general(a, b, (((1,), (1,)), ((), ())),
                           preferred_element_type=_F32)


def _front_body(nc_ref, des_ref, tw_ref, px_ref, Wnc, bnc, Wd, bd, Wt, bt,
                Wp, bp, Win, bi, h_ref):
    a = _lrelu(_dotT(nc_ref[...], Wnc[...]) + bnc[...])
    d = _lrelu(_dotT(des_ref[...], Wd[...]) + bd[...])
    t = _lrelu(_dotT(tw_ref[...], Wt[...]) + bt[...])
    p = _lrelu(_dotT(px_ref[...], Wp[...]) + bp[...])
    hcat = jnp.concatenate([a, d, t, p], axis=1)
    h_ref[...] = _lrelu(_dotT(hcat, Win[...]) + bi[...])


def _proj_body(h_ref, Wkqv, bkqv, Wkrel, Wvrel, q_ref, kc_ref, vc_ref):
    h = h_ref[...]
    kqv = _dotT(h, Wkqv[...]) + bkqv[...]
    k = kqv[:, 0:HID]
    q = kqv[:, HID:2 * HID]
    v = kqv[:, 2 * HID:3 * HID]
    q_ref[...] = q
    kc_ref[0] = _dotT(k, Wkrel[0])
    kc_ref[1] = _dotT(k, Wkrel[1])
    vc_ref[0] = _dotT(v, Wvrel[0])
    vc_ref[1] = _dotT(v, Wvrel[1])


_INV_SQRT2 = float(1.0 / np.sqrt(2.0))


def _gelu(x):
    return 0.5 * x * (1.0 + lax.erf(x * _INV_SQRT2))


def _comb1_body(h_ref, agg_ref, s_ref, Wo, bo, gv, cv, h1_ref):
    r = 1.0 / (s_ref[...] + 1e-16)
    agg = (agg_ref[0] + agg_ref[1]) * r
    gl = _gelu(agg)
    out = _dotT(gl, Wo[...]) + bo[...]
    h1_ref[...] = out * gv[...] + h_ref[...] * cv[...]


def _comb2_body(h_ref, agg_ref, s_ref, Wo, bo, gv, cv, W1, b1, W2, b2,
                out_ref, em_ref):
    r = 1.0 / (s_ref[...] + 1e-16)
    agg = (agg_ref[0] + agg_ref[1]) * r
    gl = _gelu(agg)
    out = _dotT(gl, Wo[...]) + bo[...]
    h2 = out * gv[...] + h_ref[...] * cv[...]
    em = _lrelu(_dotT(h2, W1[...]) + b1[...])
    em_ref[...] = em
    out_ref[...] = _dotT(em, W2[...]) + b2[...]


def _full(shape):
    nd = len(shape)
    return pl.BlockSpec(shape, lambda i, _n=nd: (0,) * _n)


def _rows(bshape):
    nd = len(bshape)
    if nd == 2:
        return pl.BlockSpec(bshape, lambda i: (i, 0))
    return pl.BlockSpec(bshape, lambda i: (0, i, 0))


def _tc_front(ncat, des, tw, px, Wnc, bnc, Wd, bd, Wt, bt, Wp, bp, Win, bi):
    R = 400
    grid = (N // R,)
    ins = [
        _rows((R, 17)), _rows((R, LM)), _rows((R, LM)), _rows((R, LM)),
        _full(Wnc.shape), _full(bnc.shape), _full(Wd.shape), _full(bd.shape),
        _full(Wt.shape), _full(bt.shape), _full(Wp.shape), _full(bp.shape),
        _full(Win.shape), _full(bi.shape),
    ]
    return pl.pallas_call(
        _front_body, grid=grid, in_specs=ins,
        out_specs=_rows((R, HID)),
        out_shape=jax.ShapeDtypeStruct((N, HID), _F32),
    )(ncat, des, tw, px, Wnc, bnc, Wd, bd, Wt, bt, Wp, bp, Win, bi)


def _tc_proj(h, Wkqv, bkqv, Wkrel, Wvrel):
    R = 1000
    grid = (N // R,)
    ins = [_rows((R, HID))] + [_full(w.shape) for w in
                               (Wkqv, bkqv, Wkrel, Wvrel)]
    outs = (
        pl.BlockSpec((R, HID), lambda i: (i, 0)),
        pl.BlockSpec((2, R, HID), lambda i: (0, i, 0)),
        pl.BlockSpec((2, R, HID), lambda i: (0, i, 0)),
    )
    oshape = (
        jax.ShapeDtypeStruct((N, HID), _F32),
        jax.ShapeDtypeStruct((2, N, HID), _F32),
        jax.ShapeDtypeStruct((2, N, HID), _F32),
    )
    return pl.pallas_call(
        _proj_body, grid=grid, in_specs=ins, out_specs=outs, out_shape=oshape,
    )(h, Wkqv, bkqv, Wkrel, Wvrel)


def _tc_comb1(h, aggP, s01, Wo, bo, gv, cv):
    R = 1000
    grid = (N // R,)
    ins = [
        _rows((R, HID)), _rows((2, R, HID)),
        pl.BlockSpec((R, 1), lambda i: (i, 0)),
        _full(Wo.shape), _full(bo.shape), _full(gv.shape), _full(cv.shape),
    ]
    return pl.pallas_call(
        _comb1_body, grid=grid, in_specs=ins,
        out_specs=_rows((R, HID)),
        out_shape=jax.ShapeDtypeStruct((N, HID), _F32),
    )(h, aggP, s01, Wo, bo, gv, cv)


def _tc_comb2(h, aggP, s01, Wo, bo, gv, cv, W1, b1, W2, b2):
    R = 1000
    grid = (N // R,)
    ins = [
        _rows((R, HID)), _rows((2, R, HID)),
        pl.BlockSpec((R, 1), lambda i: (i, 0)),
        _full(Wo.shape), _full(bo.shape), _full(gv.shape), _full(cv.shape),
        _full(W1.shape), _full(b1.shape), _full(W2.shape), _full(b2.shape),
    ]
    outs = (
        pl.BlockSpec((R, 2), lambda i: (i, 0)),
        pl.BlockSpec((R, 80), lambda i: (i, 0)),
    )
    oshape = (
        jax.ShapeDtypeStruct((N, 2), _F32),
        jax.ShapeDtypeStruct((N, 80), _F32),
    )
    return pl.pallas_call(
        _comb2_body, grid=grid, in_specs=ins, out_specs=outs, out_shape=oshape,
    )(h, aggP, s01, Wo, bo, gv, cv, W1, b1, W2, b2)


_MESH = plsc.VectorSubcoreMesh(core_axis_name="c", subcore_axis_name="s",
                               num_cores=NC, num_subcores=NS)
_SC_PARAMS = pltpu.CompilerParams(needs_layout_passes=False,
                                  use_tc_tiling_on_sc=False)


def _load_idx(src_hbm, dst_hbm, et_hbm, base, dstv, srcv, etv, esrcv):
    pltpu.sync_copy(dst_hbm.at[pl.ds(base, CH)], dstv)
    pltpu.sync_copy(src_hbm.at[pl.ds(base, CH)], srcv)
    pltpu.sync_copy(et_hbm.at[pl.ds(base, CH)], etv)
    for g in range(CH // 16):
        sl = pl.ds(g * 16, 16)
        esrcv[sl] = etv[sl] * N + srcv[sl]


_DSCALE = float(1.0 / np.sqrt(HID))


def _p1_body(q_hbm, k_hbm, src_hbm, dst_hbm, et_hbm, prel_hbm, a_out, m_out,
             dstv0, dstv1, srcv, etv, etv0, etv1, esrcv0, esrcv1,
             qrows0, qrows1, krows0, krows1, abuf, matb, mloc, prelv,
             semq0, semq1, semk0, semk1):
    c = lax.axis_index("c")
    s = lax.axis_index("s")
    wid = c * NS + s

    pltpu.sync_copy(prel_hbm, prelv)
    prl = prelv[pl.ds(0, 16)]
    pr0 = prl[0] * _DSCALE
    pr1 = prl[1] * _DSCALE

    def initm(i, _):
        mloc[pl.ds(i * 16, 16)] = jnp.full((16,), NEG, _F32)
        return _
    lax.fori_loop(0, NP // 16, initm, None)

    def load_fire(j, dstv, etvs, esrcv, qrows, krows, semq, semk):
        base = wid * EW + j * CH
        pltpu.sync_copy(dst_hbm.at[pl.ds(base, CH)], dstv)
        pltpu.sync_copy(src_hbm.at[pl.ds(base, CH)], srcv)
        pltpu.sync_copy(et_hbm.at[pl.ds(base, CH)], etvs)
        for g in range(CH // 16):
            sl = pl.ds(g * 16, 16)
            esrcv[sl] = etvs[sl] * N + srcv[sl]
        pltpu.async_copy(q_hbm.at[dstv], qrows, semq)
        pltpu.async_copy(k_hbm.at[esrcv], krows, semk)

    def wait_rows(dstv, esrcv, qrows, krows, semq, semk):
        pltpu.make_async_copy(q_hbm.at[dstv], qrows, semq).wait()
        pltpu.make_async_copy(k_hbm.at[esrcv], krows, semk).wait()

    def compute(j, dstv, etvs, qrows, krows):
        base = wid * EW + j * CH
        iota16 = lax.iota(jnp.int32, 16)

        def gbody(g, _):
            for jj in range(16):
                i = g * 16 + jj
                acc = qrows[i, pl.ds(0, 16)] * krows[i, pl.ds(0, 16)]
                for db in range(1, HID // 16):
                    sl = pl.ds(db * 16, 16)
                    acc = acc + qrows[i, sl] * krows[i, sl]
                matb[pl.ds(jj * 16, 16)] = acc
            r = plsc.load_gather(matb, [iota16 * 16])
            for l in range(1, 16):
                r = r + plsc.load_gather(matb, [iota16 * 16 + l])
            sl16 = pl.ds(g * 16, 16)
            etf = etvs[sl16].astype(_F32)
            avec = r * (pr0 + etf * (pr1 - pr0))
            abuf[sl16] = avec
            dvec = dstv[sl16]

            def wcond(done):
                return ~jnp.all(done)

            def wbody(done):
                cur = plsc.load_gather(mloc, [dvec])
                new = jnp.maximum(cur, avec)
                plsc.store_scatter(mloc, [dvec], new, mask=~done)
                re = plsc.load_gather(mloc, [dvec])
                return re >= avec
            lax.while_loop(wcond, wbody, jnp.zeros((16,), jnp.bool_))
            return _
        lax.fori_loop(0, CH // 16, gbody, None)
        pltpu.sync_copy(abuf, a_out.at[pl.ds(base, CH)])

    s0 = (dstv0, etv0, esrcv0, qrows0, krows0, semq0, semk0)
    s1 = (dstv1, etv1, esrcv1, qrows1, krows1, semq1, semk1)

    def lf(j, sl):
        load_fire(j, sl[0], sl[1], sl[2], sl[3], sl[4], sl[5], sl[6])

    def wt(sl):
        wait_rows(sl[0], sl[2], sl[3], sl[4], sl[5], sl[6])

    def cmp_(j, sl):
        compute(j, sl[0], sl[1], sl[3], sl[4])

    lf(0, s0)

    def pair(j2, _):
        j = 2 * j2
        wt(s0)
        lf(j + 1, s1)
        cmp_(j, s0)
        wt(s1)
        lf(j + 2, s0)
        cmp_(j + 1, s1)
        return _
    lax.fori_loop(0, (NCHUNK - 1) // 2, pair, None)
    wt(s0)
    cmp_(NCHUNK - 1, s0)

    pltpu.sync_copy(mloc, m_out.at[pl.ds(wid * NP, NP)])


_MPC = 2000


def _p2_body(a_hbm, m_hbm, v_hbm, src_hbm, dst_hbm, et_hbm,
             agg_out, s_out,
             dstv, esrcv, etmp, arow, vrows, mloc, mtmp, aggsh, ssh, sem1):
    c = lax.axis_index("c")
    s = lax.axis_index("s")
    wid = c * NS + s

    pltpu.sync_copy(m_hbm.at[pl.ds(0, NP)], mloc)

    def mpart(t, _):
        def piece(p, __):
            pltpu.sync_copy(m_hbm.at[pl.ds(t * NP + p * _MPC, _MPC)], mtmp)

            def mmax(i, ___):
                sl = pl.ds(p * _MPC + i * 16, 16)
                mloc[sl] = jnp.maximum(mloc[sl], mtmp[pl.ds(i * 16, 16)])
                return ___
            lax.fori_loop(0, _MPC // 16, mmax, None)
            return __
        lax.fori_loop(0, NP // _MPC, piece, None)
        return _
    lax.fori_loop(1, NW, mpart, None)

    for g in range(CH // 16):
        arow[pl.ds(g * 16, 16)] = jnp.zeros((16,), _F32)

    def zb(i, _):
        for db in range(HID // 16):
            vrows[i, pl.ds(db * 16, 16)] = jnp.zeros((16,), _F32)
        return _
    lax.fori_loop(0, CH, zb, None)
    nr = N // NS
    for b in range(nr // CH):
        pltpu.sync_copy(vrows, aggsh.at[pl.ds(s * nr + b * CH, CH)])
    rem = nr % CH
    pltpu.sync_copy(vrows.at[pl.ds(0, rem)],
                    aggsh.at[pl.ds(s * nr + (nr // CH) * CH, rem)])

    @pl.when(s < NS - 1)
    def _zs_full():
        for b in range(TPT // CH):
            pltpu.sync_copy(arow, ssh.at[pl.ds(s * TPT + b * CH, CH)])

    @pl.when(s == NS - 1)
    def _zs_tail():
        for b in range(TAIL // CH):
            pltpu.sync_copy(arow, ssh.at[pl.ds((NS - 1) * TPT + b * CH, CH)])
    plsc.subcore_barrier()

    def chunk(j, _):
        base = wid * EW + j * CH
        pltpu.sync_copy(dst_hbm.at[pl.ds(base, CH)], dstv)
        pltpu.sync_copy(src_hbm.at[pl.ds(base, CH)], esrcv)
        pltpu.sync_copy(et_hbm.at[pl.ds(base, CH)], etmp)
        pltpu.sync_copy(a_hbm.at[pl.ds(base, CH)], arow)
        for g in range(CH // 16):
            sl = pl.ds(g * 16, 16)
            esrcv[sl] = etmp[sl] * N + esrcv[sl]
        cp = pltpu.async_copy(v_hbm.at[esrcv], vrows, sem1)
        for g in range(CH // 16):
            sl = pl.ds(g * 16, 16)
            mv = plsc.load_gather(mloc, [dstv[sl]])
            arow[sl] = jnp.exp(arow[sl] - mv)
        pltpu.sync_copy(arow, ssh.at[dstv], add=True)
        cp.wait()
        for g in range(CH // 16):
            evg = arow[pl.ds(g * 16, 16)]
            for j in range(16):
                i = g * 16 + j
                e1 = evg[j]
                for db in range(HID // 16):
                    sl2 = pl.ds(db * 16, 16)
                    vrows[i, sl2] = vrows[i, sl2] * e1
        pltpu.sync_copy(vrows, aggsh.at[dstv], add=True)
        return _
    lax.fori_loop(0, NCHUNK, chunk, None)
    plsc.subcore_barrier()

    pltpu.sync_copy(aggsh.at[pl.ds(s * nr, nr)],
                    agg_out.at[pl.ds(c * N + s * nr, nr)])

    @pl.when(s < NS - 1)
    def _s_full():
        pltpu.sync_copy(ssh.at[pl.ds(s * TPT, TPT)],
                        s_out.at[pl.ds(c * N + s * TPT, TPT)])

    @pl.when(s == NS - 1)
    def _s_tail():
        pltpu.sync_copy(ssh.at[pl.ds((NS - 1) * TPT, TAIL)],
                        s_out.at[pl.ds(c * N + (NS - 1) * TPT, TAIL)])


_sc_pass1 = pl.kernel(
    _p1_body,
    out_type=(
        jax.ShapeDtypeStruct((E,), _F32),
        jax.ShapeDtypeStruct((NW * NP,), _F32),
    ),
    mesh=_MESH,
    scratch_types=[
        pltpu.VMEM((CH,), jnp.int32),
        pltpu.VMEM((CH,), jnp.int32),
        pltpu.VMEM((CH,), jnp.int32),
        pltpu.VMEM((CH,), jnp.int32),
        pltpu.VMEM((CH,), jnp.int32),
        pltpu.VMEM((CH,), jnp.int32),
        pltpu.VMEM((CH,), jnp.int32),
        pltpu.VMEM((CH,), jnp.int32),
        pltpu.VMEM((CH, HID), _F32),
        pltpu.VMEM((CH, HID), _F32),
        pltpu.VMEM((CH, HID), _F32),
        pltpu.VMEM((CH, HID), _F32),
        pltpu.VMEM((CH,), _F32),
        pltpu.VMEM((256,), _F32),
        pltpu.VMEM((NP,), _F32),
        pltpu.VMEM((16,), _F32),
        pltpu.SemaphoreType.DMA,
        pltpu.SemaphoreType.DMA,
        pltpu.SemaphoreType.DMA,
        pltpu.SemaphoreType.DMA,
    ],
    compiler_params=_SC_PARAMS,
)

_sc_pass2 = pl.kernel(
    _p2_body,
    out_type=(
        jax.ShapeDtypeStruct((NC * N, HID), _F32),
        jax.ShapeDtypeStruct((NC * N,), _F32),
    ),
    mesh=_MESH,
    scratch_types=[
        pltpu.VMEM((CH,), jnp.int32),
        pltpu.VMEM((CH,), jnp.int32),
        pltpu.VMEM((CH,), jnp.int32),
        pltpu.VMEM((CH,), _F32),
        pltpu.VMEM((CH, HID), _F32),
        pltpu.VMEM((NP,), _F32),
        pltpu.VMEM((_MPC,), _F32),
        pltpu.VMEM_SHARED((N, HID), _F32),
        pltpu.VMEM_SHARED((N,), _F32),
        pltpu.SemaphoreType.DMA,
    ],
    compiler_params=_SC_PARAMS,
)


def kernel(pre_x, x, edge_index, edge_type, num_prop, num_category,
           des_tensor, tweet_tensor, params):
    p = params
    f32 = _F32
    dscale = float(1.0 / np.sqrt(HID))

    Wnc = jnp.concatenate([
        jnp.concatenate([p['Wn'], jnp.zeros((32, 11), f32)], axis=1),
        jnp.concatenate([jnp.zeros((32, 6), f32), p['Wc']], axis=1),
    ], axis=0)
    bnc = jnp.concatenate([p['bn'], p['bc']])[None, :]
    ncat = jnp.concatenate([num_prop, num_category], axis=1)

    h = _tc_front(ncat, des_tensor, tweet_tensor, pre_x,
                  Wnc, bnc, p['Wd'], p['bd'][None, :], p['Wt'],
                  p['bt'][None, :], p['Wp'], p['bp'][None, :],
                  p['Win'], p['bin'][None, :])

    out = em = None
    src = edge_index[0]
    dst = edge_index[1]
    for i in (1, 2):
        Wkqv = p['Wkqv%d' % i]
        bkqv = p['bkqv%d' % i][None, :]
        Wkrel = p['Wkrel%d' % i]
        Wvrel = p['Wvrel%d' % i]
        prel16 = jnp.pad(p['prel%d' % i], (0, 14))
        g = jax.nn.sigmoid(p['skip%d' % i])
        gv = (g * jnp.ones((HID,), f32))[None, :]
        cv = ((1.0 - g) * jnp.ones((HID,), f32))[None, :]

        q, kc, vc = _tc_proj(h, Wkqv, bkqv, Wkrel, Wvrel)
        a, mP = _sc_pass1(q, kc.reshape(2 * N, HID), src, dst, edge_type,
                          prel16)
        aggF, sP = _sc_pass2(a, mP, vc.reshape(2 * N, HID),
                             src, dst, edge_type)
        aggP = aggF.reshape(NC, N, HID)
        s01 = (sP[:N] + sP[N:2 * N])[:, None]
        if i == 1:
            h = _tc_comb1(h, aggP, s01, p['Wout%d' % i],
                          p['bout%d' % i][None, :], gv, cv)
        else:
            out, em = _tc_comb2(h, aggP, s01, p['Wout%d' % i],
                                p['bout%d' % i][None, :], gv, cv,
                                p['W1'], p['b1'][None, :],
                                p['W2'], p['b2'][None, :])
    return out, em
